# Initial kernel scaffold; baseline (speedup 1.0000x reference)
import jax
import jax.numpy as jnp
from jax.experimental import pallas as pl


def kernel(x, edge_index, W1, b1, W2, b2):
    raise NotImplementedError("write your pallas kernel here")


# speedup vs baseline: 16.8125x; 16.8125x over previous
import functools

import jax
import jax.numpy as jnp
from jax import lax
from jax.experimental import pallas as pl
from jax.experimental.pallas import tpu as pltpu
from jax.experimental.pallas import tpu_sc as plsc

N = 10000
E = 320000
D = 128

NC = 2
NS = 16
NW = NC * NS
EPW = E // NW
C = 80
NCHUNK = EPW // C
NP = 10240
DPT = NP // NS
RPT = NP // NS
RZ = 128


def _build_deg_kernel(mesh):
    @functools.partial(
        pl.kernel,
        out_type=[
            jax.ShapeDtypeStruct((NC, NP), jnp.float32),
            jax.ShapeDtypeStruct((NC, NP), jnp.float32),
        ],
        mesh=mesh,
        scratch_types=[
            pltpu.VMEM((NCHUNK, 2, C), jnp.int32),
            pltpu.VMEM((C,), jnp.float32),
            pltpu.VMEM((DPT,), jnp.float32),
            pltpu.VMEM_SHARED((NP,), jnp.float32),
            pltpu.VMEM_SHARED((NP,), jnp.float32),
        ],
    )
    def deg_kernel(eidx_hbm, dego_hbm, degi_hbm,
                   idx_v, ones_v, z_v, dego_sp, degi_sp):
        c = lax.axis_index("c")
        s = lax.axis_index("s")
        wid = c * NS + s

        for j in range(C // 16):
            ones_v[pl.ds(j * 16, 16)] = jnp.ones((16,), jnp.float32)

        def zero_body(i, _):
            z_v[pl.ds(i * 16, 16)] = jnp.zeros((16,), jnp.float32)
            return 0

        lax.fori_loop(0, DPT // 16, zero_body, 0)
        pltpu.sync_copy(z_v, dego_sp.at[pl.ds(s * DPT, DPT)])
        pltpu.sync_copy(z_v, degi_sp.at[pl.ds(s * DPT, DPT)])
        plsc.subcore_barrier()

        pltpu.sync_copy(eidx_hbm.at[wid], idx_v)

        def body(i, _):
            pltpu.sync_copy(ones_v, dego_sp.at[idx_v.at[i, 0]], add=True)
            pltpu.sync_copy(ones_v, degi_sp.at[idx_v.at[i, 1]], add=True)
            return 0

        lax.fori_loop(0, NCHUNK, body, 0)
        plsc.subcore_barrier()

        pltpu.sync_copy(dego_sp.at[pl.ds(s * DPT, DPT)],
                        dego_hbm.at[c, pl.ds(s * DPT, DPT)])
        pltpu.sync_copy(degi_sp.at[pl.ds(s * DPT, DPT)],
                        degi_hbm.at[c, pl.ds(s * DPT, DPT)])

    return deg_kernel


def _build_agg_kernel(mesh):
    @functools.partial(
        pl.kernel,
        out_type=jax.ShapeDtypeStruct((NC, NP, D), jnp.float32),
        mesh=mesh,
        scratch_types=[
            pltpu.VMEM((2, C), jnp.int32),
            pltpu.VMEM((2, C), jnp.int32),
            pltpu.VMEM((C, D), jnp.float32),
            pltpu.VMEM((C, D), jnp.float32),
            pltpu.VMEM((RZ, D), jnp.float32),
            pltpu.VMEM_SHARED((NP, D), jnp.float32),
            pltpu.SemaphoreType.DMA,
            pltpu.SemaphoreType.DMA,
            pltpu.SemaphoreType.DMA,
            pltpu.SemaphoreType.DMA,
        ],
    )
    def agg_kernel(hn_hbm, eidx_hbm, out_hbm,
                   idx_a, idx_b, rows_a, rows_b, z_v, agg_sp,
                   sem_ia, sem_ib, sem_ra, sem_rb):
        c = lax.axis_index("c")
        s = lax.axis_index("s")
        wid = c * NS + s

        def zero_body(i, _):
            for j in range(D // 16):
                z_v[i, pl.ds(j * 16, 16)] = jnp.zeros((16,), jnp.float32)
            return 0

        lax.fori_loop(0, RZ, zero_body, 0)
        for k in range(RPT // RZ):
            pltpu.sync_copy(z_v, agg_sp.at[pl.ds(s * RPT + k * RZ, RZ)])
        plsc.subcore_barrier()

        def idx_start(i, ib, sem):
            pltpu.async_copy(eidx_hbm.at[wid, i], ib, sem)

        def idx_wait(ib, sem):
            pltpu.make_async_copy(eidx_hbm.at[0, 0], ib, sem).wait()

        def gather_start(ib, buf, sem):
            pltpu.async_copy(hn_hbm.at[ib.at[0]], buf, sem)

        def gather_wait(buf, sem):
            pltpu.make_async_copy(hn_hbm.at[pl.ds(0, C)], buf, sem).wait()

        def scat(ib, buf):
            pltpu.sync_copy(buf, agg_sp.at[ib.at[1]], add=True)

        idx_start(0, idx_a, sem_ia)
        idx_wait(idx_a, sem_ia)
        gather_start(idx_a, rows_a, sem_ra)
        idx_start(1, idx_b, sem_ib)

        def body(k, _):
            g = 2 * k
            idx_wait(idx_b, sem_ib)
            gather_start(idx_b, rows_b, sem_rb)
            gather_wait(rows_a, sem_ra)
            scat(idx_a, rows_a)
            idx_start(g + 2, idx_a, sem_ia)
            idx_wait(idx_a, sem_ia)
            gather_start(idx_a, rows_a, sem_ra)
            gather_wait(rows_b, sem_rb)
            scat(idx_b, rows_b)
            idx_start(g + 3, idx_b, sem_ib)
            return 0

        lax.fori_loop(0, (NCHUNK - 3) // 2, body, 0)
        idx_wait(idx_b, sem_ib)
        gather_start(idx_b, rows_b, sem_rb)
        gather_wait(rows_a, sem_ra)
        scat(idx_a, rows_a)
        idx_start(NCHUNK - 1, idx_a, sem_ia)
        idx_wait(idx_a, sem_ia)
        gather_start(idx_a, rows_a, sem_ra)
        gather_wait(rows_b, sem_rb)
        scat(idx_b, rows_b)
        gather_wait(rows_a, sem_ra)
        scat(idx_a, rows_a)

        plsc.subcore_barrier()
        pltpu.sync_copy(agg_sp.at[pl.ds(s * RPT, RPT)],
                        out_hbm.at[c, pl.ds(s * RPT, RPT)])

    return agg_kernel


@functools.cache
def _sc_kernels():
    mesh = plsc.VectorSubcoreMesh(
        core_axis_name="c", subcore_axis_name="s", num_cores=NC, num_subcores=NS
    )
    return _build_deg_kernel(mesh), _build_agg_kernel(mesh)


R = 2048
_GRID = NP // R


def _prep_body(x_ref, w1_ref, dego_ref, degi_ref,
               hn_ref, nsrc_ref, ndst_ref):
    dego = dego_ref[0] + dego_ref[1]
    degi = degi_ref[0] + degi_ref[1]
    nsrc = lax.rsqrt(jnp.maximum(dego, 1.0))
    ndst = lax.rsqrt(jnp.maximum(degi, 1.0))
    nsrc_ref[...] = nsrc
    ndst_ref[...] = ndst
    h = jnp.dot(x_ref[...], w1_ref[...], preferred_element_type=jnp.float32)
    hn_ref[...] = h * nsrc


def _mid_body(agg_ref, ndst_ref, b1_ref, w2_ref, nsrc_ref, hn_ref):
    a = agg_ref[0] + agg_ref[1]
    t = a * ndst_ref[...] + b1_ref[...]
    t = jnp.where(t >= 0.0, t, 0.01 * t)
    h = jnp.dot(t, w2_ref[...], preferred_element_type=jnp.float32)
    hn_ref[...] = h * nsrc_ref[...]


def _final_body(agg_ref, ndst_ref, b2_ref, out_ref):
    a = agg_ref[0] + agg_ref[1]
    out_ref[...] = a * ndst_ref[...] + b2_ref[...]


_row_spec = pl.BlockSpec((R, D), lambda i: (i, 0))
_col_spec = pl.BlockSpec((R, 1), lambda i: (i, 0))
_deg_spec = pl.BlockSpec((NC, R, 1), lambda i: (0, i, 0))
_agg_spec = pl.BlockSpec((NC, R, D), lambda i: (0, i, 0))
_w_spec = pl.BlockSpec((D, D), lambda i: (0, 0))
_b_spec = pl.BlockSpec((1, D), lambda i: (0, 0))

_prep_call = pl.pallas_call(
    _prep_body,
    grid=(_GRID,),
    in_specs=[_row_spec, _w_spec, _deg_spec, _deg_spec],
    out_specs=[_row_spec, _col_spec, _col_spec],
    out_shape=[
        jax.ShapeDtypeStruct((N, D), jnp.float32),
        jax.ShapeDtypeStruct((NP, 1), jnp.float32),
        jax.ShapeDtypeStruct((NP, 1), jnp.float32),
    ],
)

_mid_call = pl.pallas_call(
    _mid_body,
    grid=(_GRID,),
    in_specs=[_agg_spec, _col_spec, _b_spec, _w_spec, _col_spec],
    out_specs=_row_spec,
    out_shape=jax.ShapeDtypeStruct((N, D), jnp.float32),
)

_final_call = pl.pallas_call(
    _final_body,
    grid=(_GRID,),
    in_specs=[_agg_spec, _col_spec, _b_spec],
    out_specs=_row_spec,
    out_shape=jax.ShapeDtypeStruct((N, D), jnp.float32),
)


def kernel(x, edge_index, W1, b1, W2, b2):
    deg_kernel, agg_kernel = _sc_kernels()
    eidx = edge_index.reshape(2, NW, NCHUNK, C).transpose(1, 2, 0, 3)
    b1r = b1.reshape(1, D)
    b2r = b2.reshape(1, D)

    dego, degi = deg_kernel(eidx)
    dego = dego.reshape(NC, NP, 1)
    degi = degi.reshape(NC, NP, 1)

    hn1, nsrc, ndst = _prep_call(x, W1, dego, degi)
    agg1 = agg_kernel(hn1, eidx)
    hn2 = _mid_call(agg1, ndst, b1r, W2, nsrc)
    agg2 = agg_kernel(hn2, eidx)
    return _final_call(agg2, ndst, b2r)


# speedup vs baseline: 19.6876x; 1.1710x over previous
import functools

import jax
import jax.numpy as jnp
from jax import lax
from jax.experimental import pallas as pl
from jax.experimental.pallas import tpu as pltpu
from jax.experimental.pallas import tpu_sc as plsc

N = 10000
E = 320000
D = 128

NC = 2
NS = 16
NW = NC * NS
EPW = E // NW
C = 80
NCHUNK = EPW // C
NP = 10240
DPT = NP // NS
RPT = NP // NS
RZ = 16


def _build_deg_kernel(mesh):
    @functools.partial(
        pl.kernel,
        out_type=[
            jax.ShapeDtypeStruct((NC, NP), jnp.float32),
            jax.ShapeDtypeStruct((NC, NP), jnp.float32),
        ],
        mesh=mesh,
        scratch_types=[
            pltpu.VMEM((NCHUNK, 2, C), jnp.int32),
            pltpu.VMEM((C,), jnp.float32),
            pltpu.VMEM((DPT,), jnp.float32),
            pltpu.VMEM_SHARED((NP,), jnp.float32),
            pltpu.VMEM_SHARED((NP,), jnp.float32),
        ],
    )
    def deg_kernel(eidx_hbm, dego_hbm, degi_hbm,
                   idx_v, ones_v, z_v, dego_sp, degi_sp):
        c = lax.axis_index("c")
        s = lax.axis_index("s")
        wid = c * NS + s

        for j in range(C // 16):
            ones_v[pl.ds(j * 16, 16)] = jnp.ones((16,), jnp.float32)

        def zero_body(i, _):
            z_v[pl.ds(i * 16, 16)] = jnp.zeros((16,), jnp.float32)
            return 0

        lax.fori_loop(0, DPT // 16, zero_body, 0)
        pltpu.sync_copy(z_v, dego_sp.at[pl.ds(s * DPT, DPT)])
        pltpu.sync_copy(z_v, degi_sp.at[pl.ds(s * DPT, DPT)])
        plsc.subcore_barrier()

        pltpu.sync_copy(eidx_hbm.at[wid], idx_v)

        def body(i, _):
            pltpu.sync_copy(ones_v, dego_sp.at[idx_v.at[i, 0]], add=True)
            pltpu.sync_copy(ones_v, degi_sp.at[idx_v.at[i, 1]], add=True)
            return 0

        lax.fori_loop(0, NCHUNK, body, 0)
        plsc.subcore_barrier()

        pltpu.sync_copy(dego_sp.at[pl.ds(s * DPT, DPT)],
                        dego_hbm.at[c, pl.ds(s * DPT, DPT)])
        pltpu.sync_copy(degi_sp.at[pl.ds(s * DPT, DPT)],
                        degi_hbm.at[c, pl.ds(s * DPT, DPT)])

    return deg_kernel


def _build_agg_kernel(mesh):
    @functools.partial(
        pl.kernel,
        out_type=jax.ShapeDtypeStruct((NC, NP, D), jnp.float32),
        mesh=mesh,
        scratch_types=[
            pltpu.VMEM((2, 2, C), jnp.int32),
            pltpu.VMEM((2, 2, C), jnp.int32),
            pltpu.VMEM((C, D), jnp.float32),
            pltpu.VMEM((C, D), jnp.float32),
            pltpu.VMEM((RZ, D), jnp.float32),
            pltpu.VMEM_SHARED((NP, D), jnp.float32),
            pltpu.SemaphoreType.DMA,
            pltpu.SemaphoreType.DMA,
            pltpu.SemaphoreType.DMA,
            pltpu.SemaphoreType.DMA,
        ],
    )
    def agg_kernel(hn_hbm, eidx_hbm, out_hbm,
                   pb0, pb1, rows_a, rows_b, z_v, agg_sp,
                   sp0, sp1, sem_ra, sem_rb):
        c = lax.axis_index("c")
        s = lax.axis_index("s")
        wid = c * NS + s

        def zero_body(i, _):
            for j in range(D // 16):
                z_v[i, pl.ds(j * 16, 16)] = jnp.zeros((16,), jnp.float32)
            return 0

        lax.fori_loop(0, RZ, zero_body, 0)
        for k in range(RPT // RZ):
            pltpu.sync_copy(z_v, agg_sp.at[pl.ds(s * RPT + k * RZ, RZ)])
        plsc.subcore_barrier()

        def pair_start(i, pb, sem):
            pltpu.async_copy(eidx_hbm.at[wid, pl.ds(i, 2)], pb, sem)

        def pair_wait(pb, sem):
            pltpu.make_async_copy(eidx_hbm.at[0, pl.ds(0, 2)], pb, sem).wait()

        def gather_start(ib, buf, sem):
            pltpu.async_copy(hn_hbm.at[ib], buf, sem)

        def gather_wait(buf, sem):
            pltpu.make_async_copy(hn_hbm.at[pl.ds(0, C)], buf, sem).wait()

        def scat(ib, buf):
            pltpu.sync_copy(buf, agg_sp.at[ib], add=True)

        pair_start(0, pb0, sp0)
        pair_start(2, pb1, sp1)
        pair_wait(pb0, sp0)
        gather_start(pb0.at[0, 0], rows_a, sem_ra)

        def body(k, _):
            q = 4 * k
            gather_start(pb0.at[1, 0], rows_b, sem_rb)
            gather_wait(rows_a, sem_ra)
            scat(pb0.at[0, 1], rows_a)
            pair_wait(pb1, sp1)
            gather_start(pb1.at[0, 0], rows_a, sem_ra)
            gather_wait(rows_b, sem_rb)
            scat(pb0.at[1, 1], rows_b)
            pair_start(q + 4, pb0, sp0)
            gather_start(pb1.at[1, 0], rows_b, sem_rb)
            gather_wait(rows_a, sem_ra)
            scat(pb1.at[0, 1], rows_a)
            pair_wait(pb0, sp0)
            gather_start(pb0.at[0, 0], rows_a, sem_ra)
            gather_wait(rows_b, sem_rb)
            scat(pb1.at[1, 1], rows_b)
            pair_start(q + 6, pb1, sp1)
            return 0

        lax.fori_loop(0, (NCHUNK - 5) // 4, body, 0)
        gather_start(pb0.at[1, 0], rows_b, sem_rb)
        gather_wait(rows_a, sem_ra)
        scat(pb0.at[0, 1], rows_a)
        pair_wait(pb1, sp1)
        gather_start(pb1.at[0, 0], rows_a, sem_ra)
        gather_wait(rows_b, sem_rb)
        scat(pb0.at[1, 1], rows_b)
        pltpu.async_copy(eidx_hbm.at[wid, pl.ds(NCHUNK - 1, 1)],
                         pb0.at[pl.ds(0, 1)], sp0)
        gather_start(pb1.at[1, 0], rows_b, sem_rb)
        gather_wait(rows_a, sem_ra)
        scat(pb1.at[0, 1], rows_a)
        pltpu.make_async_copy(eidx_hbm.at[0, pl.ds(0, 1)],
                              pb0.at[pl.ds(0, 1)], sp0).wait()
        gather_start(pb0.at[0, 0], rows_a, sem_ra)
        gather_wait(rows_b, sem_rb)
        scat(pb1.at[1, 1], rows_b)
        gather_wait(rows_a, sem_ra)
        scat(pb0.at[0, 1], rows_a)

        plsc.subcore_barrier()
        pltpu.sync_copy(agg_sp.at[pl.ds(s * RPT, RPT)],
                        out_hbm.at[c, pl.ds(s * RPT, RPT)])

    return agg_kernel


@functools.cache
def _sc_kernels():
    mesh = plsc.VectorSubcoreMesh(
        core_axis_name="c", subcore_axis_name="s", num_cores=NC, num_subcores=NS
    )
    return _build_deg_kernel(mesh), _build_agg_kernel(mesh)


R = 2048
_GRID = NP // R


def _prep_body(x_ref, w1_ref, dego_ref, degi_ref,
               hn_ref, nsrc_ref, ndst_ref):
    dego = dego_ref[0] + dego_ref[1]
    degi = degi_ref[0] + degi_ref[1]
    nsrc = lax.rsqrt(jnp.maximum(dego, 1.0))
    ndst = lax.rsqrt(jnp.maximum(degi, 1.0))
    nsrc_ref[...] = nsrc
    ndst_ref[...] = ndst
    h = jnp.dot(x_ref[...], w1_ref[...], preferred_element_type=jnp.float32)
    hn_ref[...] = h * nsrc


def _mid_body(agg_ref, ndst_ref, b1_ref, w2_ref, nsrc_ref, hn_ref):
    a = agg_ref[0] + agg_ref[1]
    t = a * ndst_ref[...] + b1_ref[...]
    t = jnp.where(t >= 0.0, t, 0.01 * t)
    h = jnp.dot(t, w2_ref[...], preferred_element_type=jnp.float32)
    hn_ref[...] = h * nsrc_ref[...]


def _final_body(agg_ref, ndst_ref, b2_ref, out_ref):
    a = agg_ref[0] + agg_ref[1]
    out_ref[...] = a * ndst_ref[...] + b2_ref[...]


_row_spec = pl.BlockSpec((R, D), lambda i: (i, 0))
_col_spec = pl.BlockSpec((R, 1), lambda i: (i, 0))
_deg_spec = pl.BlockSpec((NC, R, 1), lambda i: (0, i, 0))
_agg_spec = pl.BlockSpec((NC, R, D), lambda i: (0, i, 0))
_w_spec = pl.BlockSpec((D, D), lambda i: (0, 0))
_b_spec = pl.BlockSpec((1, D), lambda i: (0, 0))

_prep_call = pl.pallas_call(
    _prep_body,
    grid=(_GRID,),
    in_specs=[_row_spec, _w_spec, _deg_spec, _deg_spec],
    out_specs=[_row_spec, _col_spec, _col_spec],
    out_shape=[
        jax.ShapeDtypeStruct((N, D), jnp.float32),
        jax.ShapeDtypeStruct((NP, 1), jnp.float32),
        jax.ShapeDtypeStruct((NP, 1), jnp.float32),
    ],
)

_mid_call = pl.pallas_call(
    _mid_body,
    grid=(_GRID,),
    in_specs=[_agg_spec, _col_spec, _b_spec, _w_spec, _col_spec],
    out_specs=_row_spec,
    out_shape=jax.ShapeDtypeStruct((N, D), jnp.float32),
)

_final_call = pl.pallas_call(
    _final_body,
    grid=(_GRID,),
    in_specs=[_agg_spec, _col_spec, _b_spec],
    out_specs=_row_spec,
    out_shape=jax.ShapeDtypeStruct((N, D), jnp.float32),
)


def kernel(x, edge_index, W1, b1, W2, b2):
    deg_kernel, agg_kernel = _sc_kernels()
    eidx = edge_index.reshape(2, NW, NCHUNK, C).transpose(1, 2, 0, 3)
    b1r = b1.reshape(1, D)
    b2r = b2.reshape(1, D)

    dego, degi = deg_kernel(eidx)
    dego = dego.reshape(NC, NP, 1)
    degi = degi.reshape(NC, NP, 1)

    hn1, nsrc, ndst = _prep_call(x, W1, dego, degi)
    agg1 = agg_kernel(hn1, eidx)
    hn2 = _mid_call(agg1, ndst, b1r, W2, nsrc)
    agg2 = agg_kernel(hn2, eidx)
    return _final_call(agg2, ndst, b2r)
